# baseline (speedup 1.0000x reference)
import functools

import jax
import jax.numpy as jnp
from jax import lax
from jax.experimental import pallas as pl
from jax.experimental.pallas import tpu as pltpu
from jax.experimental.pallas import tpu_sc as plsc

N = 10000
NP = 10240
E = 320000
D = 128
NTILES = 32
NB0 = 80
NB1 = 80
TOTROWS = 16 * (NB0 + NB1)
EP = TOTROWS * 128
KA = (6 * EP) // NTILES // 128

_mesh = plsc.VectorSubcoreMesh(core_axis_name="c", subcore_axis_name="s")


def _chunk_plan(nb):
    plan, off = [], 0
    while off < nb:
        n = min(40, nb - off)
        plan.append((off, n))
        off += n
    return plan


@functools.partial(
    pl.kernel,
    mesh=_mesh,
    out_type=jax.ShapeDtypeStruct((2, 8 * NP), jnp.float32),
    scratch_types=[
        pltpu.VMEM((KA, 128), jnp.int32),
        pltpu.VMEM((128,), jnp.float32),
        pltpu.VMEM_SHARED((8 * NP,), jnp.float32),
    ],
)
def _sc_degrees(idx_hbm, ones_hbm, zeros_hbm, out_hbm, idx_v, ones_v, acc_sh):
    c = lax.axis_index("c")
    s = lax.axis_index("s")
    wid = c * 16 + s
    per = (8 * NP) // 16
    pltpu.sync_copy(zeros_hbm, acc_sh.at[pl.ds(s * per, per)])
    pltpu.sync_copy(ones_hbm, ones_v)
    pltpu.sync_copy(idx_hbm.at[wid], idx_v)
    plsc.subcore_barrier()

    def body(k, carry):
        pltpu.sync_copy(ones_v, acc_sh.at[idx_v.at[k]], add=True)
        return carry

    lax.fori_loop(0, KA, body, 0)
    plsc.subcore_barrier()
    pltpu.sync_copy(acc_sh.at[pl.ds(s * per, per)],
                    out_hbm.at[c, pl.ds(s * per, per)])


@functools.partial(
    pl.kernel,
    mesh=_mesh,
    out_type=jax.ShapeDtypeStruct((2, 3, NP, D), jnp.float32),
    scratch_types=[
        pltpu.VMEM((40, 128), jnp.int32),
        pltpu.VMEM((40, 128), jnp.int32),
        pltpu.VMEM((128, D), jnp.float32),
        pltpu.VMEM((128, D), jnp.float32),
        pltpu.VMEM_SHARED((NP, D), jnp.float32),
        pltpu.SemaphoreType.DMA,
        pltpu.SemaphoreType.DMA,
        pltpu.SemaphoreType.DMA,
        pltpu.SemaphoreType.DMA,
    ],
)
def _sc_aggregate(xn0, xn1, xn2, srcs_hbm, dsts_hbm, zeros_hbm, out_hbm,
                  src_v, dst_v, rows_a, rows_b, acc_sh,
                  sem_a, sem_b, sem_a2, sem_b2):
    c = lax.axis_index("c")
    s = lax.axis_index("s")
    rper = NP // 16
    chunks0 = [(off, n) for off, n in _chunk_plan(NB0)]
    chunks1 = [(off, n) for off, n in _chunk_plan(NB1)]
    for r, xn in enumerate((xn0, xn1, xn2)):
        pltpu.sync_copy(zeros_hbm, acc_sh.at[pl.ds(s * rper, rper)])
        plsc.subcore_barrier()

        def wait_gather(k, rows, sm):
            pltpu.make_async_copy(xn.at[src_v.at[k]], rows, sm).wait()

        def wait_scatter(k, rows, sm):
            pltpu.make_async_copy(rows, acc_sh.at[dst_v.at[k]], sm).wait()

        def run_chunk(base, nb):
            assert nb % 2 == 0
            pltpu.sync_copy(srcs_hbm.at[r, pl.ds(base, nb)],
                            src_v.at[pl.ds(0, nb)])
            pltpu.sync_copy(dsts_hbm.at[r, pl.ds(base, nb)],
                            dst_v.at[pl.ds(0, nb)])
            pltpu.async_copy(xn.at[src_v.at[0]], rows_a, sem_a)
            pltpu.async_copy(xn.at[src_v.at[1]], rows_b, sem_b)

            def body(i, carry):
                k0 = 2 * i
                k1 = 2 * i + 1
                wait_gather(k0, rows_a, sem_a)
                pltpu.async_copy(rows_a, acc_sh.at[dst_v.at[k0]], sem_a2,
                                 add=True)
                wait_gather(k1, rows_b, sem_b)
                pltpu.async_copy(rows_b, acc_sh.at[dst_v.at[k1]], sem_b2,
                                 add=True)

                @pl.when(k0 + 2 < nb)
                def _():
                    wait_scatter(k0, rows_a, sem_a2)
                    pltpu.async_copy(xn.at[src_v.at[k0 + 2]], rows_a, sem_a)

                @pl.when(k1 + 2 < nb)
                def _():
                    wait_scatter(k1, rows_b, sem_b2)
                    pltpu.async_copy(xn.at[src_v.at[k1 + 2]], rows_b, sem_b)

                return carry

            lax.fori_loop(0, nb // 2, body, 0)
            wait_scatter(nb - 2, rows_a, sem_a2)
            wait_scatter(nb - 1, rows_b, sem_b2)

        if NB0 == NB1:
            row0 = (c * 16 + s) * NB0
            for off, n in chunks0:
                run_chunk(row0 + off, n)
        else:
            for off, n in chunks0:
                @pl.when(c == 0)
                def _(off=off, n=n):
                    run_chunk(s * NB0 + off, n)
            for off, n in chunks1:
                @pl.when(c == 1)
                def _(off=off, n=n):
                    run_chunk(16 * NB0 + s * NB1 + off, n)
        plsc.subcore_barrier()
        pltpu.sync_copy(acc_sh.at[pl.ds(s * rper, rper)],
                        out_hbm.at[c, r, pl.ds(s * rper, rper)])
        plsc.subcore_barrier()


def _tc_norm_body(x_ref, degs_ref, xn0_ref, xn1_ref, xn2_ref, ndst_ref):
    d = degs_ref[0] + degs_ref[1]
    nrm = lax.rsqrt(jnp.maximum(d, 1.0))
    ndst_ref[...] = nrm
    x = x_ref[...]
    xn0_ref[...] = x * nrm[0][:, None]
    xn1_ref[...] = x * nrm[1][:, None]
    xn2_ref[...] = x * nrm[2][:, None]


def _tc_norm(x_pad, degs):
    BN = 512
    grid = (NP // BN,)
    return pl.pallas_call(
        _tc_norm_body,
        grid=grid,
        in_specs=[
            pl.BlockSpec((BN, D), lambda i: (i, 0)),
            pl.BlockSpec((2, 8, BN), lambda i: (0, 0, i)),
        ],
        out_specs=[
            pl.BlockSpec((BN, D), lambda i: (i, 0)),
            pl.BlockSpec((BN, D), lambda i: (i, 0)),
            pl.BlockSpec((BN, D), lambda i: (i, 0)),
            pl.BlockSpec((8, BN), lambda i: (0, i)),
        ],
        out_shape=[
            jax.ShapeDtypeStruct((NP, D), jnp.float32),
            jax.ShapeDtypeStruct((NP, D), jnp.float32),
            jax.ShapeDtypeStruct((NP, D), jnp.float32),
            jax.ShapeDtypeStruct((8, NP), jnp.float32),
        ],
    )(x_pad, degs)


def _tc_final_body(p_ref, ndst_ref, w_ref, b_ref, aw1_ref, ab1_ref, aw2_ref,
                   out_ref):
    agg = p_ref[0] + p_ref[1]
    nd = ndst_ref[...]
    aw1 = aw1_ref[...]
    ab1 = ab1_ref[...]
    aw2 = aw2_ref[...]
    rel = []
    scores = []
    for r in range(3):
        h = jnp.dot(agg[r], w_ref[r], preferred_element_type=jnp.float32)
        h = h * nd[3 + r][:, None] + b_ref[r][None, :]
        rel.append(h)
        t = jnp.tanh(jnp.dot(h, aw1, preferred_element_type=jnp.float32) + ab1)
        scores.append(jnp.sum(t * aw2, axis=-1))
    m = jnp.maximum(jnp.maximum(scores[0], scores[1]), scores[2])
    e = [jnp.exp(sc - m) for sc in scores]
    denom = e[0] + e[1] + e[2]
    out = (rel[0] * e[0][:, None] + rel[1] * e[1][:, None]
           + rel[2] * e[2][:, None]) / denom[:, None]
    out_ref[...] = out


def _tc_final(p, ndst, w_stack, b_stack, aw1, ab1, aw2):
    BN = 512
    grid = (NP // BN,)
    return pl.pallas_call(
        _tc_final_body,
        grid=grid,
        in_specs=[
            pl.BlockSpec((2, 3, BN, D), lambda i: (0, 0, i, 0)),
            pl.BlockSpec((8, BN), lambda i: (0, i)),
            pl.BlockSpec((3, D, D), lambda i: (0, 0, 0)),
            pl.BlockSpec((3, D), lambda i: (0, 0)),
            pl.BlockSpec((D, D), lambda i: (0, 0)),
            pl.BlockSpec((1, D), lambda i: (0, 0)),
            pl.BlockSpec((1, D), lambda i: (0, 0)),
        ],
        out_specs=pl.BlockSpec((BN, D), lambda i: (i, 0)),
        out_shape=jax.ShapeDtypeStruct((NP, D), jnp.float32),
    )(p, ndst, w_stack, b_stack, aw1, ab1, aw2)


def kernel(x, edge_index_e0, edge_index_e1, edge_index_e2,
           W0, b0, W1, b1, W2, b2, attn_w1, attn_b1, attn_w2):
    x_pad = jnp.pad(x, ((0, NP - N), (0, 0)))

    junk = N + (jnp.arange(EP - E, dtype=jnp.int32) % 128)
    srcs, dsts, flat_parts = [], [], []
    for r, e in enumerate((edge_index_e0, edge_index_e1, edge_index_e2)):
        src = jnp.concatenate([e[0].astype(jnp.int32), junk])
        dst = jnp.concatenate([e[1].astype(jnp.int32), junk])
        srcs.append(jnp.pad(src.reshape(TOTROWS, 128), ((0, 40), (0, 0)),
                            constant_values=N))
        dsts.append(jnp.pad(dst.reshape(TOTROWS, 128), ((0, 40), (0, 0)),
                            constant_values=N))
        flat_parts.append(src + r * NP)
        flat_parts.append(dst + (3 + r) * NP)
    srcs = jnp.stack(srcs)
    dsts = jnp.stack(dsts)
    idx_a = jnp.concatenate(flat_parts).reshape(NTILES, KA, 128)

    ones128 = jnp.ones((128,), jnp.float32)
    zeros_a = jnp.zeros(((8 * NP) // 16,), jnp.float32)
    zeros_c = jnp.zeros((NP // 16, D), jnp.float32)

    degs2 = _sc_degrees(idx_a, ones128, zeros_a)
    degs = degs2.reshape(2, 8, NP)

    xn0, xn1, xn2, ndst = _tc_norm(x_pad, degs)

    p = _sc_aggregate(xn0, xn1, xn2, srcs, dsts, zeros_c)

    w_stack = jnp.stack([W0, W1, W2])
    b_stack = jnp.stack([b0, b1, b2])
    out = _tc_final(p, ndst, w_stack, b_stack, attn_w1,
                    attn_b1.reshape(1, D), attn_w2.reshape(1, D))
    return out[:N]


# speedup vs baseline: 1.0951x; 1.0951x over previous
import functools

import jax
import jax.numpy as jnp
from jax import lax
from jax.experimental import pallas as pl
from jax.experimental.pallas import tpu as pltpu
from jax.experimental.pallas import tpu_sc as plsc

N = 10000
NP = 10240
E = 320000
D = 128
NTILES = 32
NB0 = 80
NB1 = 80
TOTROWS = 16 * (NB0 + NB1)
EP = TOTROWS * 128
KA = (6 * EP) // NTILES // 128

_mesh = plsc.VectorSubcoreMesh(core_axis_name="c", subcore_axis_name="s")


def _chunk_plan(nb):
    plan, off = [], 0
    while off < nb:
        n = min(40, nb - off)
        plan.append((off, n))
        off += n
    return plan


def _tc_prep_body(e0_ref, e1_ref, e2_ref, srcs_ref, dsts_ref,
                  idxs_ref, idxd_ref):
    r = pl.program_id(0)
    i = pl.program_id(1)
    BR = 40
    lanes = lax.broadcasted_iota(jnp.int32, (1, BR, 128), 2)
    rows = lax.broadcasted_iota(jnp.int32, (1, BR, 128), 1)
    junk = N + lanes % 128
    real = (i * BR + rows) < (E // 128)

    def pick(axis):
        b0 = e0_ref[axis].reshape(1, BR, 128)
        b1 = e1_ref[axis].reshape(1, BR, 128)
        b2 = e2_ref[axis].reshape(1, BR, 128)
        v = jnp.where(r == 0, b0, jnp.where(r == 1, b1, b2))
        return jnp.where(real, v, junk)

    src = pick(0)
    dst = pick(1)
    srcs_ref[...] = src
    dsts_ref[...] = dst
    idxs_ref[...] = src + r * NP
    idxd_ref[...] = dst + (3 + r) * NP


def _tc_prep(e0, e1, e2):
    BR = 40
    grid = (3, TOTROWS // BR)
    espec = pl.BlockSpec((2, BR * 128), lambda r, i: (0, i))
    ospec = pl.BlockSpec((1, BR, 128), lambda r, i: (r, i, 0))
    oshape = jax.ShapeDtypeStruct((3, TOTROWS, 128), jnp.int32)
    pad = ((0, 0), (0, EP - E))
    return pl.pallas_call(
        _tc_prep_body,
        grid=grid,
        in_specs=[espec, espec, espec],
        out_specs=[ospec, ospec, ospec, ospec],
        out_shape=[oshape, oshape, oshape, oshape],
    )(jnp.pad(e0, pad), jnp.pad(e1, pad), jnp.pad(e2, pad))


@functools.partial(
    pl.kernel,
    mesh=_mesh,
    out_type=jax.ShapeDtypeStruct((2, 8 * NP), jnp.float32),
    scratch_types=[
        pltpu.VMEM((KA, 128), jnp.int32),
        pltpu.VMEM((128,), jnp.float32),
        pltpu.VMEM_SHARED((8 * NP,), jnp.float32),
    ],
)
def _sc_degrees(idxs_hbm, idxd_hbm, ones_hbm, zeros_hbm, out_hbm,
                idx_v, ones_v, acc_sh):
    c = lax.axis_index("c")
    s = lax.axis_index("s")
    wid = c * 16 + s
    seg = TOTROWS // 32
    per = (8 * NP) // 16
    pltpu.sync_copy(zeros_hbm, acc_sh.at[pl.ds(s * per, per)])
    pltpu.sync_copy(ones_hbm, ones_v)
    for g in range(3):
        pltpu.sync_copy(idxs_hbm.at[g, pl.ds(wid * seg, seg)],
                        idx_v.at[pl.ds(g * seg, seg)])
        pltpu.sync_copy(idxd_hbm.at[g, pl.ds(wid * seg, seg)],
                        idx_v.at[pl.ds((3 + g) * seg, seg)])
    plsc.subcore_barrier()

    def body(k, carry):
        pltpu.sync_copy(ones_v, acc_sh.at[idx_v.at[k]], add=True)
        return carry

    lax.fori_loop(0, KA, body, 0)
    plsc.subcore_barrier()
    pltpu.sync_copy(acc_sh.at[pl.ds(s * per, per)],
                    out_hbm.at[c, pl.ds(s * per, per)])


@functools.partial(
    pl.kernel,
    mesh=_mesh,
    out_type=jax.ShapeDtypeStruct((2, 3, NP, D), jnp.float32),
    scratch_types=[
        pltpu.VMEM((40, 128), jnp.int32),
        pltpu.VMEM((40, 128), jnp.int32),
        pltpu.VMEM((128, D), jnp.float32),
        pltpu.VMEM((128, D), jnp.float32),
        pltpu.VMEM_SHARED((NP, D), jnp.float32),
        pltpu.SemaphoreType.DMA,
        pltpu.SemaphoreType.DMA,
    ],
)
def _sc_aggregate(xn0, xn1, xn2, srcs_hbm, dsts_hbm, zeros_hbm, out_hbm,
                  src_v, dst_v, rows_a, rows_b, acc_sh, sem_a, sem_b):
    c = lax.axis_index("c")
    s = lax.axis_index("s")
    rper = NP // 16
    chunks0 = [(off, n) for off, n in _chunk_plan(NB0)]
    chunks1 = [(off, n) for off, n in _chunk_plan(NB1)]
    for r, xn in enumerate((xn0, xn1, xn2)):
        pltpu.sync_copy(zeros_hbm, acc_sh.at[pl.ds(s * rper, rper)])
        plsc.subcore_barrier()

        def run_chunk(base, nb):
            pltpu.sync_copy(srcs_hbm.at[r, pl.ds(base, nb)],
                            src_v.at[pl.ds(0, nb)])
            pltpu.sync_copy(dsts_hbm.at[r, pl.ds(base, nb)],
                            dst_v.at[pl.ds(0, nb)])
            pltpu.async_copy(xn.at[src_v.at[0]], rows_a, sem_a)

            def body(i, carry):
                k0 = 2 * i
                k1 = 2 * i + 1

                @pl.when(k1 < nb)
                def _():
                    pltpu.async_copy(xn.at[src_v.at[k1]], rows_b, sem_b)

                pltpu.make_async_copy(xn.at[src_v.at[k0]], rows_a,
                                      sem_a).wait()
                pltpu.sync_copy(rows_a, acc_sh.at[dst_v.at[k0]], add=True)

                @pl.when(k0 + 2 < nb)
                def _():
                    pltpu.async_copy(xn.at[src_v.at[k0 + 2]], rows_a, sem_a)

                @pl.when(k1 < nb)
                def _():
                    pltpu.make_async_copy(xn.at[src_v.at[k1]], rows_b,
                                          sem_b).wait()
                    pltpu.sync_copy(rows_b, acc_sh.at[dst_v.at[k1]],
                                    add=True)

                return carry

            lax.fori_loop(0, (nb + 1) // 2, body, 0)

        if NB0 == NB1:
            row0 = (c * 16 + s) * NB0
            for off, n in chunks0:
                run_chunk(row0 + off, n)
        else:
            for off, n in chunks0:
                @pl.when(c == 0)
                def _(off=off, n=n):
                    run_chunk(s * NB0 + off, n)
            for off, n in chunks1:
                @pl.when(c == 1)
                def _(off=off, n=n):
                    run_chunk(16 * NB0 + s * NB1 + off, n)
        plsc.subcore_barrier()
        pltpu.sync_copy(acc_sh.at[pl.ds(s * rper, rper)],
                        out_hbm.at[c, r, pl.ds(s * rper, rper)])
        plsc.subcore_barrier()


def _tc_norm_body(x_ref, degs_ref, xn0_ref, xn1_ref, xn2_ref, ndst_ref):
    d = degs_ref[0] + degs_ref[1]
    nrm = lax.rsqrt(jnp.maximum(d, 1.0))
    ndst_ref[...] = nrm
    x = x_ref[...]
    xn0_ref[...] = x * nrm[0][:, None]
    xn1_ref[...] = x * nrm[1][:, None]
    xn2_ref[...] = x * nrm[2][:, None]


def _tc_norm(x_pad, degs):
    BN = 512
    grid = (NP // BN,)
    return pl.pallas_call(
        _tc_norm_body,
        grid=grid,
        in_specs=[
            pl.BlockSpec((BN, D), lambda i: (i, 0)),
            pl.BlockSpec((2, 8, BN), lambda i: (0, 0, i)),
        ],
        out_specs=[
            pl.BlockSpec((BN, D), lambda i: (i, 0)),
            pl.BlockSpec((BN, D), lambda i: (i, 0)),
            pl.BlockSpec((BN, D), lambda i: (i, 0)),
            pl.BlockSpec((8, BN), lambda i: (0, i)),
        ],
        out_shape=[
            jax.ShapeDtypeStruct((NP, D), jnp.float32),
            jax.ShapeDtypeStruct((NP, D), jnp.float32),
            jax.ShapeDtypeStruct((NP, D), jnp.float32),
            jax.ShapeDtypeStruct((8, NP), jnp.float32),
        ],
    )(x_pad, degs)


def _tc_final_body(p_ref, ndst_ref, w_ref, b_ref, aw1_ref, ab1_ref, aw2_ref,
                   out_ref):
    agg = p_ref[0] + p_ref[1]
    nd = ndst_ref[...]
    aw1 = aw1_ref[...]
    ab1 = ab1_ref[...]
    aw2 = aw2_ref[...]
    rel = []
    scores = []
    for r in range(3):
        h = jnp.dot(agg[r], w_ref[r], preferred_element_type=jnp.float32)
        h = h * nd[3 + r][:, None] + b_ref[r][None, :]
        rel.append(h)
        t = jnp.tanh(jnp.dot(h, aw1, preferred_element_type=jnp.float32) + ab1)
        scores.append(jnp.sum(t * aw2, axis=-1))
    m = jnp.maximum(jnp.maximum(scores[0], scores[1]), scores[2])
    e = [jnp.exp(sc - m) for sc in scores]
    denom = e[0] + e[1] + e[2]
    out = (rel[0] * e[0][:, None] + rel[1] * e[1][:, None]
           + rel[2] * e[2][:, None]) / denom[:, None]
    out_ref[...] = out


def _tc_final(p, ndst, w_stack, b_stack, aw1, ab1, aw2):
    BN = 512
    grid = (NP // BN,)
    return pl.pallas_call(
        _tc_final_body,
        grid=grid,
        in_specs=[
            pl.BlockSpec((2, 3, BN, D), lambda i: (0, 0, i, 0)),
            pl.BlockSpec((8, BN), lambda i: (0, i)),
            pl.BlockSpec((3, D, D), lambda i: (0, 0, 0)),
            pl.BlockSpec((3, D), lambda i: (0, 0)),
            pl.BlockSpec((D, D), lambda i: (0, 0)),
            pl.BlockSpec((1, D), lambda i: (0, 0)),
            pl.BlockSpec((1, D), lambda i: (0, 0)),
        ],
        out_specs=pl.BlockSpec((BN, D), lambda i: (i, 0)),
        out_shape=jax.ShapeDtypeStruct((N, D), jnp.float32),
    )(p, ndst, w_stack, b_stack, aw1, ab1, aw2)


def kernel(x, edge_index_e0, edge_index_e1, edge_index_e2,
           W0, b0, W1, b1, W2, b2, attn_w1, attn_b1, attn_w2):
    x_pad = jnp.pad(x, ((0, NP - N), (0, 0)))

    srcs, dsts, idxs, idxd = _tc_prep(
        edge_index_e0.astype(jnp.int32), edge_index_e1.astype(jnp.int32),
        edge_index_e2.astype(jnp.int32))

    ones128 = jnp.ones((128,), jnp.float32)
    zeros_a = jnp.zeros(((8 * NP) // 16,), jnp.float32)
    zeros_c = jnp.zeros((NP // 16, D), jnp.float32)

    degs2 = _sc_degrees(idxs, idxd, ones128, zeros_a)
    degs = degs2.reshape(2, 8, NP)

    xn0, xn1, xn2, ndst = _tc_norm(x_pad, degs)

    p = _sc_aggregate(xn0, xn1, xn2, srcs, dsts, zeros_c)

    w_stack = jnp.stack([W0, W1, W2])
    b_stack = jnp.stack([b0, b1, b2])
    return _tc_final(p, ndst, w_stack, b_stack, attn_w1,
                     attn_b1.reshape(1, D), attn_w2.reshape(1, D))


# speedup vs baseline: 1.3083x; 1.1947x over previous
import functools

import jax
import jax.numpy as jnp
from jax import lax
from jax.experimental import pallas as pl
from jax.experimental.pallas import tpu as pltpu
from jax.experimental.pallas import tpu_sc as plsc

N = 10000
NP = 10240
E = 320000
D = 128
NTILES = 32
NB0 = 80
NB1 = 80
TOTROWS = 16 * (NB0 + NB1)
EP = TOTROWS * 128
KA = (6 * EP) // NTILES // 128

_mesh = plsc.VectorSubcoreMesh(core_axis_name="c", subcore_axis_name="s")


def _chunk_plan(nb):
    plan, off = [], 0
    while off < nb:
        n = min(40, nb - off)
        plan.append((off, n))
        off += n
    return plan


def _tc_prep_body(e0_ref, e1_ref, e2_ref, srcs_ref, dsts_ref,
                  idxs_ref, idxd_ref):
    i = pl.program_id(0)
    BR = 320
    lanes = lax.broadcasted_iota(jnp.int32, (3, BR, 128), 2)
    rows = lax.broadcasted_iota(jnp.int32, (3, BR, 128), 1)
    rel = lax.broadcasted_iota(jnp.int32, (3, BR, 128), 0)
    junk = N + lanes % 128
    real = (i * BR + rows) < (E // 128)

    def pick(axis):
        v = jnp.stack([e0_ref[axis].reshape(BR, 128),
                       e1_ref[axis].reshape(BR, 128),
                       e2_ref[axis].reshape(BR, 128)])
        return jnp.where(real, v, junk)

    src = pick(0)
    dst = pick(1)
    srcs_ref[...] = src
    dsts_ref[...] = dst
    idxs_ref[...] = src + rel * NP
    idxd_ref[...] = dst + (3 + rel) * NP


def _tc_prep(e0, e1, e2):
    BR = 320
    grid = (TOTROWS // BR,)
    espec = pl.BlockSpec((2, BR * 128), lambda i: (0, i))
    ospec = pl.BlockSpec((3, BR, 128), lambda i: (0, i, 0))
    oshape = jax.ShapeDtypeStruct((3, TOTROWS, 128), jnp.int32)
    pad = ((0, 0), (0, EP - E))
    return pl.pallas_call(
        _tc_prep_body,
        grid=grid,
        in_specs=[espec, espec, espec],
        out_specs=[ospec, ospec, ospec, ospec],
        out_shape=[oshape, oshape, oshape, oshape],
    )(jnp.pad(e0, pad), jnp.pad(e1, pad), jnp.pad(e2, pad))


@functools.partial(
    pl.kernel,
    mesh=_mesh,
    out_type=jax.ShapeDtypeStruct((2, 8 * NP), jnp.float32),
    scratch_types=[
        pltpu.VMEM((KA, 128), jnp.int32),
        pltpu.VMEM((128,), jnp.float32),
        pltpu.VMEM_SHARED((8 * NP,), jnp.float32),
    ],
)
def _sc_degrees(idxs_hbm, idxd_hbm, ones_hbm, zeros_hbm, out_hbm,
                idx_v, ones_v, acc_sh):
    c = lax.axis_index("c")
    s = lax.axis_index("s")
    wid = c * 16 + s
    seg = TOTROWS // 32
    per = (8 * NP) // 16
    pltpu.sync_copy(zeros_hbm, acc_sh.at[pl.ds(s * per, per)])
    pltpu.sync_copy(ones_hbm, ones_v)
    for g in range(3):
        pltpu.sync_copy(idxs_hbm.at[g, pl.ds(wid * seg, seg)],
                        idx_v.at[pl.ds(g * seg, seg)])
        pltpu.sync_copy(idxd_hbm.at[g, pl.ds(wid * seg, seg)],
                        idx_v.at[pl.ds((3 + g) * seg, seg)])
    plsc.subcore_barrier()

    def body(k, carry):
        pltpu.sync_copy(ones_v, acc_sh.at[idx_v.at[k]], add=True)
        return carry

    lax.fori_loop(0, KA, body, 0)
    plsc.subcore_barrier()
    pltpu.sync_copy(acc_sh.at[pl.ds(s * per, per)],
                    out_hbm.at[c, pl.ds(s * per, per)])


@functools.partial(
    pl.kernel,
    mesh=_mesh,
    out_type=jax.ShapeDtypeStruct((2, 3, NP, D), jnp.float32),
    scratch_types=[
        pltpu.VMEM((40, 128), jnp.int32),
        pltpu.VMEM((40, 128), jnp.int32),
        pltpu.VMEM((128, D), jnp.float32),
        pltpu.VMEM((128, D), jnp.float32),
        pltpu.VMEM_SHARED((NP, D), jnp.float32),
        pltpu.SemaphoreType.DMA,
        pltpu.SemaphoreType.DMA,
    ],
)
def _sc_aggregate(xn0, xn1, xn2, srcs_hbm, dsts_hbm, zeros_hbm, out_hbm,
                  src_v, dst_v, rows_a, rows_b, acc_sh, sem_a, sem_b):
    c = lax.axis_index("c")
    s = lax.axis_index("s")
    rper = NP // 16
    chunks0 = [(off, n) for off, n in _chunk_plan(NB0)]
    chunks1 = [(off, n) for off, n in _chunk_plan(NB1)]
    for r, xn in enumerate((xn0, xn1, xn2)):
        pltpu.sync_copy(zeros_hbm, acc_sh.at[pl.ds(s * rper, rper)])
        plsc.subcore_barrier()

        def run_chunk(base, nb):
            pltpu.sync_copy(srcs_hbm.at[r, pl.ds(base, nb)],
                            src_v.at[pl.ds(0, nb)])
            pltpu.sync_copy(dsts_hbm.at[r, pl.ds(base, nb)],
                            dst_v.at[pl.ds(0, nb)])
            pltpu.async_copy(xn.at[src_v.at[0]], rows_a, sem_a)

            def body(i, carry):
                k0 = 2 * i
                k1 = 2 * i + 1

                @pl.when(k1 < nb)
                def _():
                    pltpu.async_copy(xn.at[src_v.at[k1]], rows_b, sem_b)

                pltpu.make_async_copy(xn.at[src_v.at[k0]], rows_a,
                                      sem_a).wait()
                pltpu.sync_copy(rows_a, acc_sh.at[dst_v.at[k0]], add=True)

                @pl.when(k0 + 2 < nb)
                def _():
                    pltpu.async_copy(xn.at[src_v.at[k0 + 2]], rows_a, sem_a)

                @pl.when(k1 < nb)
                def _():
                    pltpu.make_async_copy(xn.at[src_v.at[k1]], rows_b,
                                          sem_b).wait()
                    pltpu.sync_copy(rows_b, acc_sh.at[dst_v.at[k1]],
                                    add=True)

                return carry

            lax.fori_loop(0, (nb + 1) // 2, body, 0)

        if NB0 == NB1:
            row0 = (c * 16 + s) * NB0
            for off, n in chunks0:
                run_chunk(row0 + off, n)
        else:
            for off, n in chunks0:
                @pl.when(c == 0)
                def _(off=off, n=n):
                    run_chunk(s * NB0 + off, n)
            for off, n in chunks1:
                @pl.when(c == 1)
                def _(off=off, n=n):
                    run_chunk(16 * NB0 + s * NB1 + off, n)
        plsc.subcore_barrier()
        pltpu.sync_copy(acc_sh.at[pl.ds(s * rper, rper)],
                        out_hbm.at[c, r, pl.ds(s * rper, rper)])
        plsc.subcore_barrier()


def _tc_norm_body(x_ref, degs_ref, xn0_ref, xn1_ref, xn2_ref, ndst_ref):
    d = degs_ref[0] + degs_ref[1]
    nrm = lax.rsqrt(jnp.maximum(d, 1.0))
    ndst_ref[...] = nrm
    x = x_ref[...]
    xn0_ref[...] = x * nrm[0][:, None]
    xn1_ref[...] = x * nrm[1][:, None]
    xn2_ref[...] = x * nrm[2][:, None]


def _tc_norm(x_pad, degs):
    BN = 512
    grid = (NP // BN,)
    return pl.pallas_call(
        _tc_norm_body,
        grid=grid,
        in_specs=[
            pl.BlockSpec((BN, D), lambda i: (i, 0)),
            pl.BlockSpec((2, 8, BN), lambda i: (0, 0, i)),
        ],
        out_specs=[
            pl.BlockSpec((BN, D), lambda i: (i, 0)),
            pl.BlockSpec((BN, D), lambda i: (i, 0)),
            pl.BlockSpec((BN, D), lambda i: (i, 0)),
            pl.BlockSpec((8, BN), lambda i: (0, i)),
        ],
        out_shape=[
            jax.ShapeDtypeStruct((NP, D), jnp.float32),
            jax.ShapeDtypeStruct((NP, D), jnp.float32),
            jax.ShapeDtypeStruct((NP, D), jnp.float32),
            jax.ShapeDtypeStruct((8, NP), jnp.float32),
        ],
    )(x_pad, degs)


def _tc_final_body(p_ref, ndst_ref, w_ref, b_ref, aw1_ref, ab1_ref, aw2_ref,
                   out_ref):
    agg = p_ref[0] + p_ref[1]
    nd = ndst_ref[...]
    aw1 = aw1_ref[...]
    ab1 = ab1_ref[...]
    aw2 = aw2_ref[...]
    rel = []
    scores = []
    for r in range(3):
        h = jnp.dot(agg[r], w_ref[r], preferred_element_type=jnp.float32)
        h = h * nd[3 + r][:, None] + b_ref[r][None, :]
        rel.append(h)
        t = jnp.tanh(jnp.dot(h, aw1, preferred_element_type=jnp.float32) + ab1)
        scores.append(jnp.sum(t * aw2, axis=-1))
    m = jnp.maximum(jnp.maximum(scores[0], scores[1]), scores[2])
    e = [jnp.exp(sc - m) for sc in scores]
    denom = e[0] + e[1] + e[2]
    out = (rel[0] * e[0][:, None] + rel[1] * e[1][:, None]
           + rel[2] * e[2][:, None]) / denom[:, None]
    out_ref[...] = out


def _tc_final(p, ndst, w_stack, b_stack, aw1, ab1, aw2):
    BN = 512
    grid = (NP // BN,)
    return pl.pallas_call(
        _tc_final_body,
        grid=grid,
        in_specs=[
            pl.BlockSpec((2, 3, BN, D), lambda i: (0, 0, i, 0)),
            pl.BlockSpec((8, BN), lambda i: (0, i)),
            pl.BlockSpec((3, D, D), lambda i: (0, 0, 0)),
            pl.BlockSpec((3, D), lambda i: (0, 0)),
            pl.BlockSpec((D, D), lambda i: (0, 0)),
            pl.BlockSpec((1, D), lambda i: (0, 0)),
            pl.BlockSpec((1, D), lambda i: (0, 0)),
        ],
        out_specs=pl.BlockSpec((BN, D), lambda i: (i, 0)),
        out_shape=jax.ShapeDtypeStruct((N, D), jnp.float32),
    )(p, ndst, w_stack, b_stack, aw1, ab1, aw2)


def kernel(x, edge_index_e0, edge_index_e1, edge_index_e2,
           W0, b0, W1, b1, W2, b2, attn_w1, attn_b1, attn_w2):
    x_pad = jnp.pad(x, ((0, NP - N), (0, 0)))

    srcs, dsts, idxs, idxd = _tc_prep(
        edge_index_e0.astype(jnp.int32), edge_index_e1.astype(jnp.int32),
        edge_index_e2.astype(jnp.int32))

    ones128 = jnp.ones((128,), jnp.float32)
    zeros_a = jnp.zeros(((8 * NP) // 16,), jnp.float32)
    zeros_c = jnp.zeros((NP // 16, D), jnp.float32)

    degs2 = _sc_degrees(idxs, idxd, ones128, zeros_a)
    degs = degs2.reshape(2, 8, NP)

    xn0, xn1, xn2, ndst = _tc_norm(x_pad, degs)

    p = _sc_aggregate(xn0, xn1, xn2, srcs, dsts, zeros_c)

    w_stack = jnp.stack([W0, W1, W2])
    b_stack = jnp.stack([b0, b1, b2])
    return _tc_final(p, ndst, w_stack, b_stack, attn_w1,
                     attn_b1.reshape(1, D), attn_w2.reshape(1, D))
